# baseline (speedup 1.0000x reference)
import jax
import jax.numpy as jnp
from jax import lax
from jax.experimental import pallas as pl
from jax.experimental.pallas import tpu as pltpu
from jax.experimental.pallas import tpu_sc as plsc

B = 16384
D = 64
V = 1000000
NW = 32
BPW = B // NW
TCOLS = (V + 127) // 128
C = 640
NB = 4
NCH = C // 16
NSC = C // 128
UROWS = B + NW
VROWS = 2 * B + 3 * NW + 32

_IOTA = lambda: lax.iota(jnp.int32, 16)


def _band(w):
    c0 = w * 244 + jnp.minimum(w, 5)
    ncols = jnp.where(w < 5, 245, jnp.where(w < 31, 244, 243))
    return c0, ncols


def _bucket(idx_hbm, idxbuf, wl_i, wl_b, done, lo, hi, sem):
    def block(blk, carry):
        off, seen = carry
        cp = pltpu.async_copy(
            idx_hbm.at[pl.ds(blk * 2048, 2048)], idxbuf, sem)
        cp.wait()

        def chunk(k, carry):
            off, seen = carry
            iv = idxbuf[pl.ds(k * 16, 16)]
            bv = blk * 2048 + k * 16 + _IOTA()
            m = (iv >= lo) & (iv < hi)
            pos = plsc.cumsum(jnp.where(m, 1, 0))
            rank = seen + pos
            keep = m & (rank > done) & (rank <= done + C)
            kcnt = plsc.all_reduce_population_count(keep)[0]
            mcnt = plsc.all_reduce_population_count(m)[0]

            @pl.when(kcnt > 0)
            def _():
                plsc.store_compressed(wl_i.at[pl.ds(off, 16)], iv, mask=keep)
                plsc.store_compressed(wl_b.at[pl.ds(off, 16)], bv, mask=keep)

            return off + kcnt, seen + mcnt

        return lax.fori_loop(0, 128, chunk, (off, seen))

    n, total = lax.fori_loop(0, B // 2048, block, (0, 0))
    return n, total


def _extract_slab(slab_ref, wl_i, wl_b, stage, n, l0, width):
    def chunk(k, carry):
        lanes = wl_i[pl.ds(k * 16, 16)] - l0
        live = (k * 16 + _IOTA()) < n
        m = (lanes >= 0) & (lanes < width) & live
        cnt = plsc.all_reduce_population_count(m)[0]

        @pl.when(cnt > 0)
        def _():
            mvec = k * 16 + _IOTA()
            for c in range(D):
                col = jnp.full((16,), c, jnp.int32)
                vals = plsc.load_gather(slab_ref, [col, lanes], mask=m)
                plsc.store_scatter(stage, [mvec, col], vals, mask=m)

        return carry

    lax.fori_loop(0, NCH, chunk, 0)


def _stage1_body(user_h, itemi_h, itemj_h, eu_h, ei_h,
                 u_out, v_out,
                 idxbuf, wl_i, wl_b, slabs, tail, stage, blist,
                 sem_i, sem_s, sem_o):
    w = lax.axis_index("s") * 2 + lax.axis_index("c")
    c0, ncols = _band(w)
    lo = c0 * 128
    hi = jnp.minimum((c0 + ncols + jnp.where(w == 31, 1, 0)) * 128, V)

    passes = (
        (user_h, eu_h, u_out, 0, B + w),
        (itemi_h, ei_h, v_out, 0, B + w),
        (itemj_h, ei_h, v_out, B + NW + 32, 2 * B + NW + 32 + w),
    )
    for idx_hbm, tab_hbm, out_hbm, roff, dummy in passes:

        def round_body(carry):
            done, _ = carry
            n, total = _bucket(idx_hbm, idxbuf, wl_i, wl_b, done, lo, hi,
                               sem_i)

            def issue(s):
                r = lax.rem(s, NB)
                lane0 = pl.multiple_of((c0 + s) * 128, 128)
                return pltpu.async_copy(
                    tab_hbm.at[:, pl.ds(lane0, 128)],
                    slabs.at[r], sem_s.at[r])

            def prime(s, carry):
                @pl.when(s < ncols)
                def _():
                    issue(s)
                return carry

            lax.fori_loop(0, NB - 1, prime, 0)

            def sbody(s, carry):
                @pl.when(s + NB - 1 < ncols)
                def _():
                    issue(s + NB - 1)

                r = lax.rem(s, NB)
                pltpu.make_async_copy(
                    tab_hbm.at[:, pl.ds(0, 128)], slabs.at[r],
                    sem_s.at[r]).wait()
                _extract_slab(slabs.at[r], wl_i, wl_b, stage, n,
                              lo + s * 128, 128)
                return carry

            lax.fori_loop(0, ncols, sbody, 0)

            @pl.when(w == 31)
            def _():
                pltpu.async_copy(
                    tab_hbm.at[:, pl.ds(TCOLS * 128 - 128, 64)], tail,
                    sem_s.at[0]).wait()
                _extract_slab(tail, wl_i, wl_b, stage, n,
                              TCOLS * 128 - 128, 64)

            def bchunk(q, carry):
                slot = q * 16 + _IOTA()
                bv = wl_b[pl.ds(q * 16, 16)]
                rows = jnp.where(slot < n, bv + roff, dummy)
                j = q // 8
                blist[j, pl.ds((q % 8) * 16, 16)] = rows
                return carry

            lax.fori_loop(0, NCH, bchunk, 0)

            for j in range(NSC):
                pltpu.async_copy(
                    stage.at[pl.ds(j * 128, 128)],
                    out_hbm.at[blist.at[j]], sem_o).wait()

            return done + n, total

        lax.while_loop(lambda c: c[0] < c[1], round_body, (0, 1))


def _stage2_body(u_h, v_h, pi_h, pj_h, bu, bi, bj, po_i, po_j, sem):
    w = lax.axis_index("s") * 2 + lax.axis_index("c")
    base = w * BPW

    for ch in range(BPW // 128):
        cb = base + ch * 128
        cu = pltpu.async_copy(u_h.at[pl.ds(cb, 128)], bu, sem.at[0])
        ci = pltpu.async_copy(v_h.at[pl.ds(cb, 128)], bi, sem.at[1])
        cj = pltpu.async_copy(v_h.at[pl.ds(B + NW + 32 + cb, 128)], bj,
                              sem.at[2])
        cu.wait()
        ci.wait()
        cj.wait()

        def group(g, carry):
            rows = g * 16 + _IOTA()
            acc_i = jnp.zeros((16,), jnp.float32)
            acc_j = jnp.zeros((16,), jnp.float32)
            for c in range(D):
                col = jnp.full((16,), c, jnp.int32)
                uc = plsc.load_gather(bu, [rows, col])
                vic = plsc.load_gather(bi, [rows, col])
                vjc = plsc.load_gather(bj, [rows, col])
                acc_i = acc_i + uc * vic
                acc_j = acc_j + uc * vjc
            po_i[pl.ds(ch * 128 + g * 16, 16)] = acc_i
            po_j[pl.ds(ch * 128 + g * 16, 16)] = acc_j
            return carry

        lax.fori_loop(0, 8, group, 0)

    pltpu.sync_copy(po_i, pi_h.at[pl.ds(base, BPW)])
    pltpu.sync_copy(po_j, pj_h.at[pl.ds(base, BPW)])


def kernel(user, item_i, item_j, embed_user, embed_item):
    mesh = plsc.VectorSubcoreMesh(core_axis_name="c", subcore_axis_name="s")
    eu_t = embed_user.T
    ei_t = embed_item.T

    stage1 = pl.kernel(
        _stage1_body,
        mesh=mesh,
        compiler_params=pltpu.CompilerParams(
            needs_layout_passes=False, use_tc_tiling_on_sc=True),
        out_type=(
            jax.ShapeDtypeStruct((UROWS, 128), jnp.float32),
            jax.ShapeDtypeStruct((VROWS, 128), jnp.float32),
        ),
        scratch_types=[
            pltpu.VMEM((2048,), jnp.int32),
            pltpu.VMEM((C + 16,), jnp.int32),
            pltpu.VMEM((C + 16,), jnp.int32),
            pltpu.VMEM((NB, D, 128), jnp.float32),
            pltpu.VMEM((D, 64), jnp.float32),
            pltpu.VMEM((C, 128), jnp.float32),
            pltpu.VMEM((NSC, 128), jnp.int32),
            pltpu.SemaphoreType.DMA,
            pltpu.SemaphoreType.DMA((NB,)),
            pltpu.SemaphoreType.DMA,
        ],
    )
    u_rows, v_rows = stage1(user, item_i, item_j, eu_t, ei_t)

    stage2 = pl.kernel(
        _stage2_body,
        mesh=mesh,
        compiler_params=pltpu.CompilerParams(
            needs_layout_passes=False, use_tc_tiling_on_sc=True),
        out_type=(
            jax.ShapeDtypeStruct((B,), jnp.float32),
            jax.ShapeDtypeStruct((B,), jnp.float32),
        ),
        scratch_types=[
            pltpu.VMEM((128, 128), jnp.float32),
            pltpu.VMEM((128, 128), jnp.float32),
            pltpu.VMEM((128, 128), jnp.float32),
            pltpu.VMEM((BPW,), jnp.float32),
            pltpu.VMEM((BPW,), jnp.float32),
            pltpu.SemaphoreType.DMA((3,)),
        ],
    )
    return stage2(u_rows, v_rows)


# speedup vs baseline: 1.1254x; 1.1254x over previous
import jax
import jax.numpy as jnp
from jax import lax
from jax.experimental import pallas as pl
from jax.experimental.pallas import tpu as pltpu
from jax.experimental.pallas import tpu_sc as plsc

B = 16384
D = 64
V = 1000000
NW = 32
BPW = B // NW
BCOLS = 244
NWIN = 61
C = 1024
SB = 32
SBC = 64
PC = 64
JOFF = B + NW + 32
UROWS = B + NW
VROWS = JOFF + B + NW

_IOTA = lambda: lax.iota(jnp.int32, 16)


def _bucket(idx_hbm, idxbuf, wl_v, wl_b, done, lo, hi, sem):
    def block(blk, carry):
        pltpu.async_copy(
            idx_hbm.at[pl.ds(blk * 1024, 1024)], idxbuf, sem).wait()

        def chunk(k, carry):
            off, seen = carry
            iv = idxbuf[pl.ds(k * 16, 16)]
            bv = blk * 1024 + k * 16 + _IOTA()
            m = (iv >= lo) & (iv < hi)
            rank = seen + plsc.cumsum(jnp.where(m, 1, 0))
            keep = m & (rank > done) & (rank <= done + C)
            kcnt = plsc.all_reduce_population_count(keep)[0]
            mcnt = plsc.all_reduce_population_count(m)[0]

            @pl.when(kcnt > 0)
            def _():
                plsc.store_compressed(wl_v.at[pl.ds(off, 16)], iv, mask=keep)
                plsc.store_compressed(wl_b.at[pl.ds(off, 16)], bv, mask=keep)

            return off + kcnt, seen + mcnt

        return lax.fori_loop(0, 64, chunk, carry)

    return lax.fori_loop(0, B // 1024, block, (0, 0))


def _split(wl_v, n, lo, sb_l, sb_m):
    sent = jnp.full((16,), jnp.int32(1 << 28), jnp.int32)

    def initc(t, carry):
        sb_l[pl.ds(t * 16, 16)] = sent
        return carry

    lax.fori_loop(0, SB * SBC // 16, initc, 0)

    def chunk(q, offs):
        lanes = wl_v[pl.ds(q * 16, 16)] - lo
        live = (q * 16 + _IOTA()) < n
        sub = jnp.clip(lanes >> 10, 0, SB - 1)
        mvec = q * 16 + _IOTA()
        out = []
        for s in range(SB):
            off_s = offs[s]
            msk = live & (sub == s)
            cnt = plsc.all_reduce_population_count(msk)[0]

            @pl.when((cnt > 0) & (off_s <= SBC - 16))
            def _(s=s, off_s=off_s, msk=msk):
                plsc.store_compressed(
                    sb_l.at[pl.ds(s * SBC + off_s, 16)], lanes, mask=msk)
                plsc.store_compressed(
                    sb_m.at[pl.ds(s * SBC + off_s, 16)], mvec, mask=msk)

            out.append(off_s + cnt)
        return tuple(out)

    offs = lax.fori_loop(0, (n + 15) // 16, chunk, (0,) * SB)
    ovf = offs[0] > (SBC - 16)
    for s in range(1, SB):
        ovf = ovf | (offs[s] > (SBC - 16))
    return ovf


def _stage1_body(user_h, itemi_h, itemj_h, eu_h, ei_h,
                 u_out, v_out,
                 idxbuf, slabs, tailbuf, ministage, blistm,
                 wl_vA, wl_bA, sb_lA, sb_mA, stageA, blistA, pend_lA, pend_mA,
                 wl_vB, wl_bB, sb_lB, sb_mB, stageB, blistB, pend_lB, pend_mB,
                 sem_i, sem_s, sem_oA, sem_oB, sem_m):
    w = lax.axis_index("s") * 2 + lax.axis_index("c")
    c0 = w * BCOLS
    lo = c0 * 128
    hi = jnp.where(w == 31, V, (c0 + BCOLS) * 128)
    nwin = NWIN + jnp.where(w == 31, 1, 0)

    setA = (wl_vA, wl_bA, sb_lA, sb_mA, stageA, blistA, pend_lA, pend_mA,
            sem_oA)
    setB = (wl_vB, wl_bB, sb_lB, sb_mB, stageB, blistB, pend_lB, pend_mB,
            sem_oB)

    def run_pass(tab_hbm, lists):

        def issue_window(k):
            par = lax.rem(k, 2)
            for j in range(4):
                lane0 = pl.multiple_of((c0 + k * 4 + j) * 128, 128)
                pltpu.async_copy(
                    tab_hbm.at[:, pl.ds(lane0, 128)],
                    slabs.at[par * 4 + j], sem_s.at[par * 4 + j])

        def extract_chunk(rel, msk, par, gmask_rows, stage_ref, rowbase):
            slot = par * 4 + jnp.clip(rel >> 7, 0, 3)
            lin = rel & 127
            for c in range(D):
                col = jnp.full((16,), c, jnp.int32)
                vals = plsc.load_gather(slabs, [slot, col, lin], mask=msk)
                plsc.store_scatter(stage_ref, [rowbase + _IOTA(), col], vals,
                                   mask=msk)

        def round_body(carry):
            states = []
            flat = carry
            for li, (idx_hbm, out_hbm, roff, dummy, sset) in enumerate(lists):
                done = flat[2 * li]
                (wl_v, wl_b, sb_l, sb_m, stage, blist, pend_l, pend_m,
                 sem_o) = sset
                n, total = _bucket(idx_hbm, idxbuf, wl_v, wl_b, done, lo, hi,
                                   sem_i)
                ovf = _split(wl_v, n, lo, sb_l, sb_m)
                states.append((n, total, ovf))

            for kk in range(2):
                issue_window(jnp.int32(kk))

            def wbody(k, carry):
                par = lax.rem(k, 2)
                for j in range(4):
                    pltpu.make_async_copy(
                        tab_hbm.at[:, pl.ds(0, 128)],
                        slabs.at[par * 4 + j], sem_s.at[par * 4 + j]).wait()
                win_l0 = k * 512
                sw = k >> 1

                for li, (idx_hbm, out_hbm, roff, dummy, sset) in enumerate(
                        lists):
                    (wl_v, wl_b, sb_l, sb_m, stage, blist, pend_l, pend_m,
                     sem_o) = sset
                    n, total, ovf = states[li]
                    dum = dummy

                    def pchunk(q, poff):
                        lanes = sb_l[pl.ds(sw * SBC + q * 16, 16)]
                        mv = sb_m[pl.ds(sw * SBC + q * 16, 16)]
                        rel = lanes - win_l0
                        msk = (rel >= 0) & (rel < 512)
                        cnt = plsc.all_reduce_population_count(msk)[0]

                        @pl.when((cnt > 0) & (poff <= PC - 16))
                        def _():
                            plsc.store_compressed(
                                pend_l.at[pl.ds(poff, 16)], rel, mask=msk)
                            plsc.store_compressed(
                                pend_m.at[pl.ds(poff, 16)], mv, mask=msk)

                        return poff + cnt

                    poff = lax.fori_loop(0, SBC // 16, pchunk, 0)
                    use_slow = ovf | (poff > PC - 16)
                    neff = jnp.where(use_slow, 0, poff)

                    @pl.when(k >= 2)
                    def _():
                        pltpu.make_async_copy(
                            stage.at[par, pl.ds(0, PC)],
                            out_hbm.at[blist.at[par]], sem_o.at[par]).wait()

                    def fblock(pb, carry):
                        livemask = (pb * 16 + _IOTA()) < neff
                        rel = pend_l[pl.ds(pb * 16, 16)]
                        extract_chunk(rel, livemask, par, None,
                                      stage.at[par], pb * 16)
                        return carry

                    lax.fori_loop(0, (neff + 15) // 16, fblock, 0)

                    for q in range(PC // 16):
                        slotv = q * 16 + _IOTA()
                        mvq = pend_m[pl.ds(q * 16, 16)]
                        bq = plsc.load_gather(wl_b, [jnp.clip(mvq, 0, C + 15)])
                        rows = jnp.where(slotv < neff, bq + roff, dum)
                        blist[par, pl.ds(q * 16, 16)] = rows

                    pltpu.async_copy(
                        stage.at[par, pl.ds(0, PC)],
                        out_hbm.at[blist.at[par]], sem_o.at[par])

                    @pl.when(use_slow)
                    def _():
                        def sc(q, carry):
                            iv = wl_v[pl.ds(q * 16, 16)]
                            live = (q * 16 + _IOTA()) < n
                            rel = iv - lo - win_l0
                            msk = live & (rel >= 0) & (rel < 512)
                            cnt = plsc.all_reduce_population_count(msk)[0]

                            @pl.when(cnt > 0)
                            def _():
                                extract_chunk(rel, msk, par, None,
                                              ministage, 0)
                                bq = wl_b[pl.ds(q * 16, 16)]
                                rowsm = jnp.where(msk, bq + roff, dum)
                                blistm[0, pl.ds(0, 16)] = rowsm
                                pltpu.async_copy(
                                    ministage.at[pl.ds(0, 16)],
                                    out_hbm.at[blistm.at[0]], sem_m).wait()

                            return carry

                        lax.fori_loop(0, (n + 15) // 16, sc, 0)

                @pl.when(k + 2 < nwin)
                def _():
                    issue_window(k + 2)

                return carry

            lax.fori_loop(0, nwin, wbody, 0)

            @pl.when(w == 31)
            def _():
                pltpu.async_copy(
                    tab_hbm.at[:, pl.ds((V // 128) * 128, 64)],
                    tailbuf, sem_s.at[0]).wait()
                tail_l0 = (V // 128) * 128 - lo
                for li, (idx_hbm, out_hbm, roff, dummy, sset) in enumerate(
                        lists):
                    (wl_v, wl_b, sb_l, sb_m, stage, blist, pend_l, pend_m,
                     sem_o) = sset
                    n, total, ovf = states[li]
                    dum = dummy

                    def tc(q, carry):
                        iv = wl_v[pl.ds(q * 16, 16)]
                        live = (q * 16 + _IOTA()) < n
                        rel = iv - lo - tail_l0
                        msk = live & (rel >= 0) & (rel < 64)
                        cnt = plsc.all_reduce_population_count(msk)[0]

                        @pl.when(cnt > 0)
                        def _():
                            lin = rel & 63
                            for c in range(D):
                                col = jnp.full((16,), c, jnp.int32)
                                vals = plsc.load_gather(
                                    tailbuf, [col, lin], mask=msk)
                                plsc.store_scatter(
                                    ministage, [_IOTA(), col], vals, mask=msk)
                            bq = wl_b[pl.ds(q * 16, 16)]
                            rowsm = jnp.where(msk, bq + roff, dum)
                            blistm[0, pl.ds(0, 16)] = rowsm
                            pltpu.async_copy(
                                ministage.at[pl.ds(0, 16)],
                                out_hbm.at[blistm.at[0]], sem_m).wait()

                        return carry

                    lax.fori_loop(0, (n + 15) // 16, tc, 0)

            out_carry = []
            for li, (idx_hbm, out_hbm, roff, dummy, sset) in enumerate(lists):
                (wl_v, wl_b, sb_l, sb_m, stage, blist, pend_l, pend_m,
                 sem_o) = sset
                for par in (0, 1):
                    pltpu.make_async_copy(
                        stage.at[par, pl.ds(0, PC)],
                        out_hbm.at[blist.at[par]], sem_o.at[par]).wait()
                n, total, ovf = states[li]
                done = flat[2 * li]
                out_carry.extend([done + n, total])
            return tuple(out_carry)

        def cond(carry):
            more = carry[0] < carry[1]
            for li in range(1, len(lists)):
                more = more | (carry[2 * li] < carry[2 * li + 1])
            return more

        lax.while_loop(cond, round_body, (0, 1) * len(lists))

    run_pass(eu_h, ((user_h, u_out, 0, B + w, setA),))
    run_pass(ei_h, ((itemi_h, v_out, 0, B + w, setA),
                    (itemj_h, v_out, JOFF, JOFF + B + w, setB)))


def _stage2_body(u_h, v_h, pi_h, pj_h, bu, bi, bj, po_i, po_j, sem):
    w = lax.axis_index("s") * 2 + lax.axis_index("c")
    base = w * BPW

    for ch in range(BPW // 128):
        cb = base + ch * 128
        cu = pltpu.async_copy(u_h.at[pl.ds(cb, 128)], bu, sem.at[0])
        ci = pltpu.async_copy(v_h.at[pl.ds(cb, 128)], bi, sem.at[1])
        cj = pltpu.async_copy(v_h.at[pl.ds(JOFF + cb, 128)], bj, sem.at[2])
        cu.wait()
        ci.wait()
        cj.wait()

        def group(g, carry):
            rows = g * 16 + _IOTA()
            acc_i = jnp.zeros((16,), jnp.float32)
            acc_j = jnp.zeros((16,), jnp.float32)
            for c in range(D):
                col = jnp.full((16,), c, jnp.int32)
                uc = plsc.load_gather(bu, [rows, col])
                vic = plsc.load_gather(bi, [rows, col])
                vjc = plsc.load_gather(bj, [rows, col])
                acc_i = acc_i + uc * vic
                acc_j = acc_j + uc * vjc
            po_i[pl.ds(ch * 128 + g * 16, 16)] = acc_i
            po_j[pl.ds(ch * 128 + g * 16, 16)] = acc_j
            return carry

        lax.fori_loop(0, 8, group, 0)

    pltpu.sync_copy(po_i, pi_h.at[pl.ds(base, BPW)])
    pltpu.sync_copy(po_j, pj_h.at[pl.ds(base, BPW)])


def kernel(user, item_i, item_j, embed_user, embed_item):
    mesh = plsc.VectorSubcoreMesh(core_axis_name="c", subcore_axis_name="s")
    eu_t = embed_user.T
    ei_t = embed_item.T

    def listset():
        return [
            pltpu.VMEM((C + 32,), jnp.int32),
            pltpu.VMEM((C + 32,), jnp.int32),
            pltpu.VMEM((SB * SBC,), jnp.int32),
            pltpu.VMEM((SB * SBC,), jnp.int32),
            pltpu.VMEM((2, PC, 128), jnp.float32),
            pltpu.VMEM((2, PC), jnp.int32),
            pltpu.VMEM((SBC + 16,), jnp.int32),
            pltpu.VMEM((SBC + 16,), jnp.int32),
        ]

    stage1 = pl.kernel(
        _stage1_body,
        mesh=mesh,
        compiler_params=pltpu.CompilerParams(
            needs_layout_passes=False, use_tc_tiling_on_sc=True),
        out_type=(
            jax.ShapeDtypeStruct((UROWS, 128), jnp.float32),
            jax.ShapeDtypeStruct((VROWS, 128), jnp.float32),
        ),
        scratch_types=[
            pltpu.VMEM((1024,), jnp.int32),
            pltpu.VMEM((8, D, 128), jnp.float32),
            pltpu.VMEM((D, 64), jnp.float32),
            pltpu.VMEM((16, 128), jnp.float32),
            pltpu.VMEM((1, 16), jnp.int32),
        ] + listset() + listset() + [
            pltpu.SemaphoreType.DMA,
            pltpu.SemaphoreType.DMA((8,)),
            pltpu.SemaphoreType.DMA((2,)),
            pltpu.SemaphoreType.DMA((2,)),
            pltpu.SemaphoreType.DMA,
        ],
    )

    u_rows, v_rows = stage1(user, item_i, item_j, eu_t, ei_t)

    stage2 = pl.kernel(
        _stage2_body,
        mesh=mesh,
        compiler_params=pltpu.CompilerParams(
            needs_layout_passes=False, use_tc_tiling_on_sc=True),
        out_type=(
            jax.ShapeDtypeStruct((B,), jnp.float32),
            jax.ShapeDtypeStruct((B,), jnp.float32),
        ),
        scratch_types=[
            pltpu.VMEM((128, 128), jnp.float32),
            pltpu.VMEM((128, 128), jnp.float32),
            pltpu.VMEM((128, 128), jnp.float32),
            pltpu.VMEM((BPW,), jnp.float32),
            pltpu.VMEM((BPW,), jnp.float32),
            pltpu.SemaphoreType.DMA((3,)),
        ],
    )
    return stage2(u_rows, v_rows)
